# baseline (speedup 1.0000x reference)
import functools

import jax
import jax.numpy as jnp
from jax import lax
from jax.experimental import pallas as pl
from jax.experimental.pallas import tpu as pltpu
from jax.experimental.pallas import tpu_sc as plsc

NC = 2
NS = 16
NW = NC * NS
EB = 128
NB = 80
E_PAD = NW * NB * EB
NACC = 79 * EB
RPT = NACC // NS
DUMMY = 10016
HID1 = 64

_MESH = plsc.VectorSubcoreMesh(core_axis_name="c", subcore_axis_name="s")


DEG_W = 128
NBD = E_PAD // (NS * EB)


@functools.partial(
    pl.kernel,
    mesh=_MESH,
    out_type=jax.ShapeDtypeStruct((NC, NACC, DEG_W), jnp.float32),
    scratch_types=[
        pltpu.VMEM((NBD, EB), jnp.int32),
        pltpu.VMEM((EB, DEG_W), jnp.float32),
        pltpu.VMEM_SHARED((NACC, DEG_W), jnp.float32),
    ],
)
def _deg_kernel(idx_hbm, ones_hbm, zeros_hbm, out_hbm, idx_v, ones_v, acc):
  c = lax.axis_index("c")
  s = lax.axis_index("s")
  r0 = s * RPT
  pltpu.sync_copy(zeros_hbm.at[pl.ds(r0, RPT)], acc.at[pl.ds(r0, RPT)])
  pltpu.sync_copy(ones_hbm, ones_v)
  pltpu.sync_copy(idx_hbm.at[c, s], idx_v)
  plsc.subcore_barrier()

  def body(j, carry):
    pltpu.sync_copy(ones_v, acc.at[idx_v.at[j]], add=True)
    return carry

  lax.fori_loop(0, NBD, body, 0)
  plsc.subcore_barrier()
  pltpu.sync_copy(acc.at[pl.ds(r0, RPT)], out_hbm.at[c, pl.ds(r0, RPT)])


D_SC = 128


@functools.partial(
    pl.kernel,
    mesh=_MESH,
    out_type=jax.ShapeDtypeStruct((NC, NACC, D_SC), jnp.float32),
    scratch_types=[
        pltpu.VMEM((NB, EB), jnp.int32),
        pltpu.VMEM((NB, EB), jnp.int32),
        pltpu.VMEM((EB, D_SC), jnp.float32),
        pltpu.VMEM_SHARED((NACC, D_SC), jnp.float32),
        pltpu.SemaphoreType.DMA,
    ],
)
def _scatter_kernel(z_hbm, src_hbm, dst_hbm, zeros_hbm, out_hbm,
                    src_v, dst_v, rows_v, acc, sem):
  c = lax.axis_index("c")
  s = lax.axis_index("s")
  wid = s * NC + c
  r0 = s * RPT
  pltpu.sync_copy(zeros_hbm.at[pl.ds(r0, RPT)], acc.at[pl.ds(r0, RPT)])
  pltpu.sync_copy(src_hbm.at[wid], src_v)
  pltpu.sync_copy(dst_hbm.at[wid], dst_v)
  plsc.subcore_barrier()

  def body(j, carry):
    pltpu.async_copy(z_hbm.at[src_v.at[j]], rows_v, sem).wait()
    pltpu.sync_copy(rows_v, acc.at[dst_v.at[j]], add=True)
    return carry

  lax.fori_loop(0, NB, body, 0)
  plsc.subcore_barrier()
  pltpu.sync_copy(acc.at[pl.ds(r0, RPT)], out_hbm.at[c, pl.ds(r0, RPT)])


def _tc_prep_body(x_ref, w_ref, ho_ref, hi_ref, z_ref, ro_ref, ri_ref):
  deg_o = jnp.maximum(ho_ref[:, 0:1], 1.0)
  deg_i = jnp.maximum(hi_ref[:, 0:1], 1.0)
  ro = lax.rsqrt(deg_o)
  ri = lax.rsqrt(deg_i)
  ro_ref[...] = ro
  ri_ref[...] = ri
  z = jnp.dot(x_ref[...] * ro, w_ref[...],
              preferred_element_type=jnp.float32)
  z_ref[...] = jnp.concatenate(
      [z, jnp.zeros((z.shape[0], D_SC - z.shape[1]), jnp.float32)], axis=1)


def _tc_mid_body(g_ref, ri_ref, ro_ref, w_ref, z_ref):
  g = g_ref[0, :, :HID1] + g_ref[1, :, :HID1]
  h1 = jnp.maximum(g * ri_ref[...], 0.0)
  z = jnp.dot(h1 * ro_ref[...], w_ref[...],
              preferred_element_type=jnp.float32)
  z_ref[...] = jnp.concatenate(
      [z, jnp.zeros((z.shape[0], D_SC - z.shape[1]), jnp.float32)], axis=1)


def _tc_out_body(g_ref, ri_ref, o_ref):
  ncls = o_ref.shape[1]
  o_ref[...] = (g_ref[0, :, :ncls] + g_ref[1, :, :ncls]) * ri_ref[...]


def kernel(x, edge_index, W1, W2):
  n_nodes = x.shape[0]
  n_edges = edge_index.shape[1]
  d_in = x.shape[1]
  h1 = W1.shape[1]
  n_cls = W2.shape[1]
  pad = E_PAD - n_edges

  src = edge_index[0]
  dst = edge_index[1]
  src_g = jnp.concatenate(
      [src, jnp.zeros((pad,), jnp.int32)]).reshape(NW, NB, EB)
  src_d = jnp.concatenate(
      [src, jnp.full((pad,), DUMMY, jnp.int32)]).reshape(NW, NB, EB)
  dst_p = jnp.concatenate(
      [dst, jnp.full((pad,), DUMMY, jnp.int32)]).reshape(NW, NB, EB)

  xp = jnp.pad(x, ((0, NACC - n_nodes), (0, 0)))
  ones_deg = jnp.ones((EB, DEG_W), jnp.float32)
  zeros_deg = jnp.zeros((NACC, DEG_W), jnp.float32)
  zeros_w = jnp.zeros((NACC, D_SC), jnp.float32)

  idx_deg = jnp.stack([src_d.reshape(NS, NBD, EB),
                       dst_p.reshape(NS, NBD, EB)])
  degs = _deg_kernel(idx_deg, ones_deg, zeros_deg)
  deg_o = degs[0]
  deg_i = degs[1]

  z1, r_out, r_in = pl.pallas_call(
      _tc_prep_body,
      out_shape=[
          jax.ShapeDtypeStruct((NACC, D_SC), jnp.float32),
          jax.ShapeDtypeStruct((NACC, 1), jnp.float32),
          jax.ShapeDtypeStruct((NACC, 1), jnp.float32),
      ],
  )(xp, W1, deg_o, deg_i)

  g1 = _scatter_kernel(z1, src_g, dst_p, zeros_w)

  z2 = pl.pallas_call(
      _tc_mid_body,
      out_shape=jax.ShapeDtypeStruct((NACC, D_SC), jnp.float32),
  )(g1, r_in, r_out, W2)

  g2 = _scatter_kernel(z2, src_g, dst_p, zeros_w)

  out = pl.pallas_call(
      _tc_out_body,
      out_shape=jax.ShapeDtypeStruct((NACC, n_cls), jnp.float32),
  )(g2, r_in)

  return out[:n_nodes]


# speedup vs baseline: 1.3819x; 1.3819x over previous
import functools

import jax
import jax.numpy as jnp
from jax import lax
from jax.experimental import pallas as pl
from jax.experimental.pallas import tpu as pltpu
from jax.experimental.pallas import tpu_sc as plsc

NC = 2
NS = 16
NW = NC * NS
EB = 128
NB = 79
E_PAD = NW * NB * EB
NACC = 79 * EB
RPT = NACC // NS
DUMMY = 10016
HID1 = 64

_MESH = plsc.VectorSubcoreMesh(core_axis_name="c", subcore_axis_name="s")


DEG_W = 128
NBD = E_PAD // (NS * EB)


@functools.partial(
    pl.kernel,
    mesh=_MESH,
    out_type=jax.ShapeDtypeStruct((NC, NACC, DEG_W), jnp.float32),
    scratch_types=[
        pltpu.VMEM((NBD, EB), jnp.int32),
        pltpu.VMEM((EB, DEG_W), jnp.float32),
        pltpu.VMEM_SHARED((NACC, DEG_W), jnp.float32),
    ],
)
def _deg_kernel(idx_hbm, ones_hbm, zeros_hbm, out_hbm, idx_v, ones_v, acc):
  c = lax.axis_index("c")
  s = lax.axis_index("s")
  r0 = s * RPT
  pltpu.sync_copy(zeros_hbm.at[pl.ds(r0, RPT)], acc.at[pl.ds(r0, RPT)])
  pltpu.sync_copy(ones_hbm, ones_v)
  pltpu.sync_copy(idx_hbm.at[c, s], idx_v)
  plsc.subcore_barrier()

  def body(j, carry):
    pltpu.sync_copy(ones_v, acc.at[idx_v.at[j]], add=True)
    return carry

  lax.fori_loop(0, NBD, body, 0)
  plsc.subcore_barrier()
  pltpu.sync_copy(acc.at[pl.ds(r0, RPT)], out_hbm.at[c, pl.ds(r0, RPT)])


D_SC = 128


@functools.partial(
    pl.kernel,
    mesh=_MESH,
    out_type=jax.ShapeDtypeStruct((NC, NACC, D_SC), jnp.float32),
    scratch_types=[
        pltpu.VMEM((NB, EB), jnp.int32),
        pltpu.VMEM((NB, EB), jnp.int32),
        pltpu.VMEM((EB, D_SC), jnp.float32),
        pltpu.VMEM_SHARED((NACC, D_SC), jnp.float32),
        pltpu.SemaphoreType.DMA,
    ],
)
def _scatter_kernel(z_hbm, src_hbm, dst_hbm, zeros_hbm, out_hbm,
                    src_v, dst_v, rows_v, acc, sem):
  c = lax.axis_index("c")
  s = lax.axis_index("s")
  wid = s * NC + c
  r0 = s * RPT
  pltpu.sync_copy(zeros_hbm.at[pl.ds(r0, RPT)], acc.at[pl.ds(r0, RPT)])
  pltpu.sync_copy(src_hbm.at[wid], src_v)
  pltpu.sync_copy(dst_hbm.at[wid], dst_v)
  plsc.subcore_barrier()

  def body(j, carry):
    pltpu.async_copy(z_hbm.at[src_v.at[j]], rows_v, sem).wait()
    pltpu.sync_copy(rows_v, acc.at[dst_v.at[j]], add=True)
    return carry

  lax.fori_loop(0, NB, body, 0)
  plsc.subcore_barrier()
  pltpu.sync_copy(acc.at[pl.ds(r0, RPT)], out_hbm.at[c, pl.ds(r0, RPT)])


def _tc_prep_body(x_ref, w_ref, ho_ref, hi_ref, z_ref, ro_ref, ri_ref):
  deg_o = jnp.maximum(ho_ref[:, 0:1], 1.0)
  deg_i = jnp.maximum(hi_ref[:, 0:1], 1.0)
  ro = lax.rsqrt(deg_o)
  ri = lax.rsqrt(deg_i)
  ro_ref[...] = ro
  ri_ref[...] = ri
  z = jnp.dot(x_ref[...] * ro, w_ref[...],
              preferred_element_type=jnp.float32)
  z_ref[...] = jnp.concatenate(
      [z, jnp.zeros((z.shape[0], D_SC - z.shape[1]), jnp.float32)], axis=1)


def _tc_mid_body(g_ref, ri_ref, ro_ref, w_ref, z_ref):
  g = g_ref[0, :, :HID1] + g_ref[1, :, :HID1]
  h1 = jnp.maximum(g * ri_ref[...], 0.0)
  z = jnp.dot(h1 * ro_ref[...], w_ref[...],
              preferred_element_type=jnp.float32)
  z_ref[...] = jnp.concatenate(
      [z, jnp.zeros((z.shape[0], D_SC - z.shape[1]), jnp.float32)], axis=1)


def _tc_out_body(g_ref, ri_ref, o_ref):
  ncls = o_ref.shape[1]
  o_ref[...] = (g_ref[0, :, :ncls] + g_ref[1, :, :ncls]) * ri_ref[...]


def kernel(x, edge_index, W1, W2):
  n_nodes = x.shape[0]
  n_edges = edge_index.shape[1]
  d_in = x.shape[1]
  h1 = W1.shape[1]
  n_cls = W2.shape[1]
  pad = E_PAD - n_edges

  src = edge_index[0]
  dst = edge_index[1]
  src_g = jnp.concatenate(
      [src, jnp.zeros((pad,), jnp.int32)]).reshape(NW, NB, EB)
  src_d = jnp.concatenate(
      [src, jnp.full((pad,), DUMMY, jnp.int32)]).reshape(NW, NB, EB)
  dst_p = jnp.concatenate(
      [dst, jnp.full((pad,), DUMMY, jnp.int32)]).reshape(NW, NB, EB)

  xp = jnp.pad(x, ((0, NACC - n_nodes), (0, 0)))
  ones_deg = jnp.ones((EB, DEG_W), jnp.float32)
  zeros_deg = jnp.zeros((NACC, DEG_W), jnp.float32)
  zeros_w = jnp.zeros((NACC, D_SC), jnp.float32)

  idx_deg = jnp.stack([src_d.reshape(NS, NBD, EB),
                       dst_p.reshape(NS, NBD, EB)])
  degs = _deg_kernel(idx_deg, ones_deg, zeros_deg)
  deg_o = degs[0]
  deg_i = degs[1]

  z1, r_out, r_in = pl.pallas_call(
      _tc_prep_body,
      out_shape=[
          jax.ShapeDtypeStruct((NACC, D_SC), jnp.float32),
          jax.ShapeDtypeStruct((NACC, 1), jnp.float32),
          jax.ShapeDtypeStruct((NACC, 1), jnp.float32),
      ],
  )(xp, W1, deg_o, deg_i)

  g1 = _scatter_kernel(z1, src_g, dst_p, zeros_w)

  z2 = pl.pallas_call(
      _tc_mid_body,
      out_shape=jax.ShapeDtypeStruct((NACC, D_SC), jnp.float32),
  )(g1, r_in, r_out, W2)

  g2 = _scatter_kernel(z2, src_g, dst_p, zeros_w)

  out = pl.pallas_call(
      _tc_out_body,
      out_shape=jax.ShapeDtypeStruct((NACC, n_cls), jnp.float32),
  )(g2, r_in)

  return out[:n_nodes]


# speedup vs baseline: 2.2125x; 1.6011x over previous
import functools

import jax
import jax.numpy as jnp
from jax import lax
from jax.experimental import pallas as pl
from jax.experimental.pallas import tpu as pltpu
from jax.experimental.pallas import tpu_sc as plsc

NC = 2
NS = 16
NW = NC * NS
EB = 128
NB = 79
E_PAD = NW * NB * EB
NACC = 79 * EB
RPT = NACC // NS
DUMMY = 10016
HID1 = 64

_MESH = plsc.VectorSubcoreMesh(core_axis_name="c", subcore_axis_name="s")


DEG_W = 128
NBD = E_PAD // (NS * EB)


@functools.partial(
    pl.kernel,
    mesh=_MESH,
    out_type=jax.ShapeDtypeStruct((NC, NACC, DEG_W), jnp.float32),
    scratch_types=[
        pltpu.VMEM((NBD, EB), jnp.int32),
        pltpu.VMEM((EB, DEG_W), jnp.float32),
        pltpu.VMEM_SHARED((NACC, DEG_W), jnp.float32),
    ],
)
def _deg_kernel(idx_hbm, ones_hbm, zeros_hbm, out_hbm, idx_v, ones_v, acc):
  c = lax.axis_index("c")
  s = lax.axis_index("s")
  r0 = s * RPT
  pltpu.sync_copy(zeros_hbm.at[pl.ds(r0, RPT)], acc.at[pl.ds(r0, RPT)])
  pltpu.sync_copy(ones_hbm, ones_v)
  pltpu.sync_copy(idx_hbm.at[c, s], idx_v)
  plsc.subcore_barrier()

  def body(j, carry):
    pltpu.sync_copy(ones_v, acc.at[idx_v.at[j]], add=True)
    return carry

  lax.fori_loop(0, NBD, body, 0)
  plsc.subcore_barrier()
  pltpu.sync_copy(acc.at[pl.ds(r0, RPT)], out_hbm.at[c, pl.ds(r0, RPT)])


D_SC = 128


@functools.partial(
    pl.kernel,
    mesh=_MESH,
    out_type=jax.ShapeDtypeStruct((NC, NACC, D_SC), jnp.float32),
    scratch_types=[
        pltpu.VMEM((NB, EB), jnp.int32),
        pltpu.VMEM((NB, EB), jnp.int32),
        pltpu.VMEM((EB, D_SC), jnp.float32),
        pltpu.VMEM_SHARED((NACC, D_SC), jnp.float32),
        pltpu.SemaphoreType.DMA,
    ],
)
def _scatter_kernel(z_hbm, src_hbm, dst_hbm, zeros_hbm, out_hbm,
                    src_v, dst_v, rows_v, acc, sem):
  c = lax.axis_index("c")
  s = lax.axis_index("s")
  wid = s * NC + c
  r0 = s * RPT
  pltpu.sync_copy(zeros_hbm.at[pl.ds(r0, RPT)], acc.at[pl.ds(r0, RPT)])
  pltpu.sync_copy(src_hbm.at[wid], src_v)
  pltpu.sync_copy(dst_hbm.at[wid], dst_v)
  plsc.subcore_barrier()

  def body(j, carry):
    pltpu.async_copy(z_hbm.at[src_v.at[j]], rows_v, sem).wait()
    pltpu.sync_copy(rows_v, acc.at[dst_v.at[j]], add=True)
    return carry

  lax.fori_loop(0, NB, body, 0)
  plsc.subcore_barrier()
  pltpu.sync_copy(acc.at[pl.ds(r0, RPT)], out_hbm.at[c, pl.ds(r0, RPT)])


def _tc_prep_body(x_ref, w_ref, ho_ref, hi_ref, z_ref, ro_ref, ri_ref):
  deg_o = jnp.maximum(ho_ref[:, 0:1], 1.0)
  deg_i = jnp.maximum(hi_ref[:, 0:1], 1.0)
  ro = lax.rsqrt(deg_o)
  ri = lax.rsqrt(deg_i)
  ro_ref[...] = ro
  ri_ref[...] = ri
  z = jnp.dot(x_ref[...] * ro, w_ref[...],
              preferred_element_type=jnp.float32)
  z_ref[...] = jnp.concatenate(
      [z, jnp.zeros((z.shape[0], D_SC - z.shape[1]), jnp.float32)], axis=1)


def _tc_mid_body(g_ref, ri_ref, ro_ref, w_ref, z_ref):
  g = g_ref[0, :, :HID1] + g_ref[1, :, :HID1]
  h1 = jnp.maximum(g * ri_ref[...], 0.0)
  z = jnp.dot(h1 * ro_ref[...], w_ref[...],
              preferred_element_type=jnp.float32)
  z_ref[...] = jnp.concatenate(
      [z, jnp.zeros((z.shape[0], D_SC - z.shape[1]), jnp.float32)], axis=1)


def _tc_out_body(g_ref, ri_ref, o_ref):
  ncls = o_ref.shape[1]
  o_ref[...] = (g_ref[0, :, :ncls] + g_ref[1, :, :ncls]) * ri_ref[...]


def kernel(x, edge_index, W1, W2):
  n_nodes = x.shape[0]
  n_edges = edge_index.shape[1]
  d_in = x.shape[1]
  h1 = W1.shape[1]
  n_cls = W2.shape[1]
  pad = E_PAD - n_edges

  src = edge_index[0]
  dst = edge_index[1]
  pad_rows = n_nodes + (
      jax.lax.iota(jnp.int32, pad) % jnp.int32(NACC - n_nodes))
  src_g = jnp.concatenate([src, pad_rows]).reshape(NW, NB, EB)
  src_d = src_g
  dst_p = jnp.concatenate([dst, pad_rows]).reshape(NW, NB, EB)

  xp = jnp.pad(x, ((0, NACC - n_nodes), (0, 0)))
  ones_deg = jnp.ones((EB, DEG_W), jnp.float32)
  zeros_deg = jnp.zeros((NACC, DEG_W), jnp.float32)
  zeros_w = jnp.zeros((NACC, D_SC), jnp.float32)

  idx_deg = jnp.stack([src_d.reshape(NS, NBD, EB),
                       dst_p.reshape(NS, NBD, EB)])
  degs = _deg_kernel(idx_deg, ones_deg, zeros_deg)
  deg_o = degs[0]
  deg_i = degs[1]

  z1, r_out, r_in = pl.pallas_call(
      _tc_prep_body,
      out_shape=[
          jax.ShapeDtypeStruct((NACC, D_SC), jnp.float32),
          jax.ShapeDtypeStruct((NACC, 1), jnp.float32),
          jax.ShapeDtypeStruct((NACC, 1), jnp.float32),
      ],
  )(xp, W1, deg_o, deg_i)

  g1 = _scatter_kernel(z1, src_g, dst_p, zeros_w)

  z2 = pl.pallas_call(
      _tc_mid_body,
      out_shape=jax.ShapeDtypeStruct((NACC, D_SC), jnp.float32),
  )(g1, r_in, r_out, W2)

  g2 = _scatter_kernel(z2, src_g, dst_p, zeros_w)

  out = pl.pallas_call(
      _tc_out_body,
      out_shape=jax.ShapeDtypeStruct((NACC, n_cls), jnp.float32),
  )(g2, r_in)

  return out[:n_nodes]
